# baseline (speedup 1.0000x reference)
import jax
import jax.numpy as jnp
from jax import lax
from jax.experimental import pallas as pl
from jax.experimental.pallas import tpu as pltpu
from jax.experimental.pallas import tpu_sc as plsc

NUM_CLASSES = 21
ALPHA = 0.75
B, R = 16, 100000
NC, NS, LANES = 2, 16, 16
NW = NC * NS
W = 128
NBLK = 99968 // W
R_IN = NBLK * W
GRPS = W // LANES
NT = -(-NBLK // NW)
NT_EVEN = NT + (NT % 2)
SC_B0 = 8
BW_TC = 6400


def _focal_body(ct_hbm, lbl_hbm, tail_hbm, out_hbm, class_v, lbl_v, out_v, tail_v, in_sems, out_sems):
    wid = lax.axis_index("s") * NC + lax.axis_index("c")

    def slab_r0(t):
        g = jnp.minimum(wid + NW * t, NBLK - 1)
        return pl.multiple_of(g * W, W)

    def start_in(b, t):
        r0 = slab_r0(t)
        pltpu.async_copy(
            ct_hbm.at[:, pl.ds(SC_B0, 8), pl.ds(r0, W)], class_v.at[b], in_sems.at[b]
        )
        pltpu.async_copy(
            lbl_hbm.at[pl.ds(SC_B0, 8), pl.ds(r0, W)], lbl_v.at[b], in_sems.at[b]
        )

    def wait_in(b):
        pltpu.make_async_copy(
            ct_hbm.at[:, pl.ds(0, 8), pl.ds(0, W)], class_v.at[b], in_sems.at[b]
        ).wait()
        pltpu.make_async_copy(
            lbl_hbm.at[pl.ds(0, 8), pl.ds(0, W)], lbl_v.at[b], in_sems.at[b]
        ).wait()

    def start_out(b, t):
        r0 = slab_r0(t)
        for s in range(8):
            off = pl.multiple_of(s * R + r0, 8)
            pltpu.async_copy(
                out_v.at[b, s], out_hbm.at[pl.ds(off, W)], out_sems.at[b]
            )

    def wait_out(b):
        for s in range(8):
            pltpu.make_async_copy(
                out_v.at[b, s], out_hbm.at[pl.ds(0, W)], out_sems.at[b]
            ).wait()

    def compute(b):
        for s in range(8):

            def grp(i, c2, s=s):
                c0 = i * LANES
                lbl = lbl_v[b, s, pl.ds(c0, LANES)]
                p = class_v[b, 0, s, pl.ds(c0, LANES)]
                for k in range(1, NUM_CLASSES):
                    p = jnp.where(lbl == k, class_v[b, k, s, pl.ds(c0, LANES)], p)
                a = jnp.where(lbl == 0, 1.0 - ALPHA, ALPHA).astype(jnp.float32)
                r = 1.0 - p
                out_v[b, s, pl.ds(c0, LANES)] = (a * r) * r
                return c2

            lax.fori_loop(0, GRPS, grp, 0, unroll=4)

    start_in(0, 0)

    def pair(tp, carry):
        t0 = 2 * tp
        for par in range(2):
            t = t0 + par
            nxt = 1 - par

            @pl.when(t + 1 < NT_EVEN)
            def _():
                start_in(nxt, t + 1)

            wait_in(par)

            @pl.when(tp > 0)
            def _():
                wait_out(par)

            compute(par)
            start_out(par, t)
        return carry

    lax.fori_loop(0, NT_EVEN // 2, pair, 0)
    wait_out(0)
    wait_out(1)

    @pl.when(wid == 0)
    def _tail():
        pltpu.sync_copy(tail_hbm, tail_v)
        for b in range(8):
            pltpu.sync_copy(
                tail_v.at[pl.ds(32 * b, 32)],
                out_hbm.at[pl.ds(pl.multiple_of(b * R + R_IN, 8), 32)],
            )


_focal_call = pl.kernel(
    _focal_body,
    out_type=jax.ShapeDtypeStruct((8 * R,), jnp.float32),
    mesh=plsc.VectorSubcoreMesh(core_axis_name="c", subcore_axis_name="s"),
    scratch_types=[
        pltpu.VMEM((2, NUM_CLASSES, 8, W), jnp.float32),
        pltpu.VMEM((2, 8, W), jnp.int32),
        pltpu.VMEM((2, 8, W), jnp.float32),
        pltpu.VMEM((8 * 32,), jnp.float32),
        pltpu.SemaphoreType.DMA((2,)),
        pltpu.SemaphoreType.DMA((2,)),
    ],
)


def _tc_body(ct_ref, lbl_ref, out_ref):
    lbl = lbl_ref[...]
    p = ct_ref[0]
    for k in range(1, NUM_CLASSES):
        p = jnp.where(lbl == k, ct_ref[k], p)
    a = jnp.where(lbl == 0, 1.0 - ALPHA, ALPHA).astype(jnp.float32)
    r = 1.0 - p
    out_ref[...] = (a * r) * r


_tc_call = pl.pallas_call(
    _tc_body,
    out_shape=jax.ShapeDtypeStruct((8, R), jnp.float32),
    grid=(-(-R // BW_TC),),
    in_specs=[
        pl.BlockSpec((NUM_CLASSES, 8, BW_TC), lambda g: (0, 0, g)),
        pl.BlockSpec((8, BW_TC), lambda g: (0, g)),
    ],
    out_specs=pl.BlockSpec((8, BW_TC), lambda g: (0, g)),
)


def kernel(classification, labels):
    lbl = labels.astype(jnp.int32)
    ct = jnp.transpose(classification, (2, 0, 1))
    lbl_t = lbl[SC_B0:, R_IN:]
    p_t = jnp.take_along_axis(
        classification[SC_B0:, R_IN:, :], lbl_t[:, :, None], axis=2
    )[:, :, 0]
    a_t = jnp.where(lbl_t == 0, 1.0 - ALPHA, ALPHA).astype(jnp.float32)
    tail = (a_t * (1.0 - p_t) ** 2).reshape(-1)
    sc_flat = _focal_call(ct, lbl, tail)
    tc2d = _tc_call(ct, lbl)
    return jnp.concatenate([tc2d.reshape(-1), sc_flat])


# speedup vs baseline: 1.0761x; 1.0761x over previous
import jax
import jax.numpy as jnp
from jax import lax
from jax.experimental import pallas as pl
from jax.experimental.pallas import tpu as pltpu
from jax.experimental.pallas import tpu_sc as plsc

NUM_CLASSES = 21
ALPHA = 0.75
B, R = 16, 100000
NC, NS, LANES = 2, 16, 16
NW = NC * NS
W = 128
NBLK = 99968 // W
R_IN = NBLK * W
GRPS = W // LANES
NT = -(-NBLK // NW)
NT_EVEN = NT + (NT % 2)
SC_B0 = 8
BW_TC = 6400


def _focal_body(ct_hbm, lbl_hbm, tail_hbm, out_hbm, class_v, lbl_v, out_v, tail_v, in_sems, out_sems):
    wid = lax.axis_index("s") * NC + lax.axis_index("c")

    def slab_r0(t):
        g = jnp.minimum(wid + NW * t, NBLK - 1)
        return pl.multiple_of(g * W, W)

    def start_in(b, t):
        r0 = slab_r0(t)
        pltpu.async_copy(
            ct_hbm.at[:, pl.ds(SC_B0, 8), pl.ds(r0, W)], class_v.at[b], in_sems.at[b]
        )
        pltpu.async_copy(
            lbl_hbm.at[pl.ds(SC_B0, 8), pl.ds(r0, W)], lbl_v.at[b], in_sems.at[b]
        )

    def wait_in(b):
        pltpu.make_async_copy(
            ct_hbm.at[:, pl.ds(0, 8), pl.ds(0, W)], class_v.at[b], in_sems.at[b]
        ).wait()
        pltpu.make_async_copy(
            lbl_hbm.at[pl.ds(0, 8), pl.ds(0, W)], lbl_v.at[b], in_sems.at[b]
        ).wait()

    def start_out(b, t):
        r0 = slab_r0(t)
        for s in range(8):
            off = pl.multiple_of(s * R + r0, 8)
            pltpu.async_copy(
                out_v.at[b, s], out_hbm.at[pl.ds(off, W)], out_sems.at[b]
            )

    def wait_out(b):
        for s in range(8):
            pltpu.make_async_copy(
                out_v.at[b, s], out_hbm.at[pl.ds(0, W)], out_sems.at[b]
            ).wait()

    def compute(b):
        for s in range(8):

            def grp(i, c2, s=s):
                c0 = i * LANES
                lbl = lbl_v[b, s, pl.ds(c0, LANES)]
                p = class_v[b, 0, s, pl.ds(c0, LANES)]
                for k in range(1, NUM_CLASSES):
                    p = jnp.where(lbl == k, class_v[b, k, s, pl.ds(c0, LANES)], p)
                a = jnp.where(lbl == 0, 1.0 - ALPHA, ALPHA).astype(jnp.float32)
                r = 1.0 - p
                out_v[b, s, pl.ds(c0, LANES)] = (a * r) * r
                return c2

            lax.fori_loop(0, GRPS, grp, 0, unroll=2)

    start_in(0, 0)

    def pair(tp, carry):
        t0 = 2 * tp
        for par in range(2):
            t = t0 + par
            nxt = 1 - par

            @pl.when(t + 1 < NT_EVEN)
            def _():
                start_in(nxt, t + 1)

            wait_in(par)

            @pl.when(tp > 0)
            def _():
                wait_out(par)

            compute(par)
            start_out(par, t)
        return carry

    lax.fori_loop(0, NT_EVEN // 2, pair, 0)
    wait_out(0)
    wait_out(1)

    @pl.when(wid == 0)
    def _tail():
        pltpu.sync_copy(tail_hbm, tail_v)
        for b in range(8):
            pltpu.sync_copy(
                tail_v.at[pl.ds(32 * b, 32)],
                out_hbm.at[pl.ds(pl.multiple_of(b * R + R_IN, 8), 32)],
            )


_focal_call = pl.kernel(
    _focal_body,
    out_type=jax.ShapeDtypeStruct((8 * R,), jnp.float32),
    mesh=plsc.VectorSubcoreMesh(core_axis_name="c", subcore_axis_name="s"),
    scratch_types=[
        pltpu.VMEM((2, NUM_CLASSES, 8, W), jnp.float32),
        pltpu.VMEM((2, 8, W), jnp.int32),
        pltpu.VMEM((2, 8, W), jnp.float32),
        pltpu.VMEM((8 * 32,), jnp.float32),
        pltpu.SemaphoreType.DMA((2,)),
        pltpu.SemaphoreType.DMA((2,)),
    ],
)


def _tc_body(ct_ref, lbl_ref, out_ref):
    lbl = lbl_ref[...]
    p = ct_ref[0]
    for k in range(1, NUM_CLASSES):
        p = jnp.where(lbl == k, ct_ref[k], p)
    a = jnp.where(lbl == 0, 1.0 - ALPHA, ALPHA).astype(jnp.float32)
    r = 1.0 - p
    out_ref[...] = (a * r) * r


_tc_call = pl.pallas_call(
    _tc_body,
    out_shape=jax.ShapeDtypeStruct((8, R), jnp.float32),
    grid=(-(-R // BW_TC),),
    in_specs=[
        pl.BlockSpec((NUM_CLASSES, 8, BW_TC), lambda g: (0, 0, g)),
        pl.BlockSpec((8, BW_TC), lambda g: (0, g)),
    ],
    out_specs=pl.BlockSpec((8, BW_TC), lambda g: (0, g)),
)


def kernel(classification, labels):
    lbl = labels.astype(jnp.int32)
    ct = jnp.transpose(classification, (2, 0, 1))
    lbl_t = lbl[SC_B0:, R_IN:]
    p_t = jnp.take_along_axis(
        classification[SC_B0:, R_IN:, :], lbl_t[:, :, None], axis=2
    )[:, :, 0]
    a_t = jnp.where(lbl_t == 0, 1.0 - ALPHA, ALPHA).astype(jnp.float32)
    tail = (a_t * (1.0 - p_t) ** 2).reshape(-1)
    sc_flat = _focal_call(ct, lbl, tail)
    tc2d = _tc_call(ct, lbl)
    return jnp.concatenate([tc2d.reshape(-1), sc_flat])
